# baseline (speedup 1.0000x reference)
import functools
import math

import jax
import jax.numpy as jnp
from jax import lax
from jax.experimental import pallas as pl
from jax.experimental.pallas import tpu as pltpu
from jax.experimental.pallas import tpu_sc as plsc

_M = 0.5
_COS_M = math.cos(_M)
_SIN_M = math.sin(_M)
_TH = math.cos(math.pi - _M)
_MM = math.sin(math.pi - _M) * _M
_EPS = 1e-07

_ROWS = 1024
_COLS = 100000


_NC = 2
_NS = 16
_NW = _NC * _NS
_RPW = _ROWS // _NW
_L = 16


def _rsqrt16(z):
    b = lax.bitcast_convert_type(z, jnp.int32)
    y = lax.bitcast_convert_type(
        jnp.int32(0x5F3759DF) - lax.shift_right_arithmetic(b, 1), jnp.float32)
    for _ in range(3):
        y = y * (1.5 - 0.5 * z * y * y)
    return y


def _sc_phi_body(cos_hbm, lbl_hbm, phi_hbm, lbl_v, idx_v, val_v, sem):
    wid = lax.axis_index("s") * _NC + lax.axis_index("c")
    base = wid * _RPW
    pltpu.sync_copy(lbl_hbm.at[pl.ds(base, _RPW)], lbl_v)
    for j in range(_RPW // _L):
        lbl = lbl_v[pl.ds(j * _L, _L)]
        rows = (base + j * _L) + lax.iota(jnp.int32, _L)
        idx_v[pl.ds(j * _L, _L)] = (
            (lax.shift_left((lax.shift_right_logical(lbl, 3) << 3)
                            + lax.shift_right_logical(rows, 7), 10))
            + lax.shift_left(lbl & 7, 7) + (rows & 127))
    pltpu.async_copy(cos_hbm.at[idx_v], val_v, sem).wait()
    for j in range(_RPW // _L):
        x = val_v[pl.ds(j * _L, _L)]
        x = jnp.minimum(jnp.maximum(x, -1.0 + _EPS), 1.0 - _EPS)
        z = 1.0 - x * x
        sine = z * _rsqrt16(z)
        phi = x * _COS_M - sine * _SIN_M
        phi = jnp.where(x > _TH, phi, x - _MM)
        val_v[pl.ds(j * _L, _L)] = phi
    pltpu.sync_copy(val_v, phi_hbm.at[pl.ds(base, _RPW)])


@functools.cache
def _sc_phi():
    return functools.partial(
        pl.kernel,
        out_type=jax.ShapeDtypeStruct((_ROWS,), jnp.float32),
        mesh=plsc.VectorSubcoreMesh(
            core_axis_name="c", subcore_axis_name="s",
            num_cores=_NC, num_subcores=_NS),
        scratch_types=[
            pltpu.VMEM((_RPW,), jnp.int32),
            pltpu.VMEM((_RPW,), jnp.int32),
            pltpu.VMEM((_RPW,), jnp.float32),
            pltpu.SemaphoreType.DMA,
        ],
    )(_sc_phi_body)


_BCLS = 1000


def _tc_body(s_ref, x_ref, lbl_ref, phi_ref, o_ref):
    cls = pl.program_id(0) * _BCLS + lax.broadcasted_iota(
        jnp.int32, (_BCLS, _ROWS), 0)
    mask = cls == lbl_ref[...]
    o_ref[...] = jnp.where(mask, phi_ref[...], x_ref[...]) * s_ref[0]


def kernel(cosine, labels, s):
    lbl = labels.astype(jnp.int32)
    ct = cosine.T
    flat_phys = jnp.transpose(
        ct.reshape(_COLS // 8, 8, _ROWS // 128, 128), (0, 2, 1, 3)
    ).reshape(-1)
    phi = _sc_phi()(flat_phys, lbl)
    s_arr = jnp.asarray(s, jnp.float32).reshape(1)
    out_t = pl.pallas_call(
        _tc_body,
        grid=(_COLS // _BCLS,),
        in_specs=[
            pl.BlockSpec(memory_space=pltpu.SMEM),
            pl.BlockSpec((_BCLS, _ROWS), lambda i: (i, 0)),
            pl.BlockSpec((1, _ROWS), lambda i: (0, 0)),
            pl.BlockSpec((1, _ROWS), lambda i: (0, 0)),
        ],
        out_specs=pl.BlockSpec((_BCLS, _ROWS), lambda i: (i, 0)),
        out_shape=jax.ShapeDtypeStruct((_COLS, _ROWS), jnp.float32),
        compiler_params=pltpu.CompilerParams(
            dimension_semantics=("arbitrary",)),
    )(s_arr, ct, lbl.reshape(1, _ROWS), phi.reshape(1, _ROWS))
    return out_t.T


# speedup vs baseline: 1.0115x; 1.0115x over previous
import functools
import math

import jax
import jax.numpy as jnp
from jax import lax
from jax.experimental import pallas as pl
from jax.experimental.pallas import tpu as pltpu
from jax.experimental.pallas import tpu_sc as plsc

_M = 0.5
_COS_M = math.cos(_M)
_SIN_M = math.sin(_M)
_TH = math.cos(math.pi - _M)
_MM = math.sin(math.pi - _M) * _M
_EPS = 1e-07

_ROWS = 1024
_COLS = 100000


_NC = 2
_NS = 16
_NW = _NC * _NS
_RPW = _ROWS // _NW
_L = 16


def _rsqrt16(z):
    b = lax.bitcast_convert_type(z, jnp.int32)
    y = lax.bitcast_convert_type(
        jnp.int32(0x5F3759DF) - lax.shift_right_arithmetic(b, 1), jnp.float32)
    for _ in range(3):
        y = y * (1.5 - 0.5 * z * y * y)
    return y


def _sc_phi_body(cos_hbm, lbl_hbm, phi_hbm, lbl_v, idx_v, val_v, sem):
    wid = lax.axis_index("s") * _NC + lax.axis_index("c")
    base = wid * _RPW
    pltpu.sync_copy(lbl_hbm.at[pl.ds(base, _RPW)], lbl_v)
    for j in range(_RPW // _L):
        lbl = lbl_v[pl.ds(j * _L, _L)]
        rows = (base + j * _L) + lax.iota(jnp.int32, _L)
        idx_v[pl.ds(j * _L, _L)] = (
            (lax.shift_left((lax.shift_right_logical(lbl, 3) << 3)
                            + lax.shift_right_logical(rows, 7), 10))
            + lax.shift_left(lbl & 7, 7) + (rows & 127))
    pltpu.async_copy(cos_hbm.at[idx_v], val_v, sem).wait()
    for j in range(_RPW // _L):
        x = val_v[pl.ds(j * _L, _L)]
        x = jnp.minimum(jnp.maximum(x, -1.0 + _EPS), 1.0 - _EPS)
        z = 1.0 - x * x
        sine = z * _rsqrt16(z)
        phi = x * _COS_M - sine * _SIN_M
        phi = jnp.where(x > _TH, phi, x - _MM)
        val_v[pl.ds(j * _L, _L)] = phi
    pltpu.sync_copy(val_v, phi_hbm.at[pl.ds(base, _RPW)])


@functools.cache
def _sc_phi():
    return functools.partial(
        pl.kernel,
        out_type=jax.ShapeDtypeStruct((_ROWS,), jnp.float32),
        mesh=plsc.VectorSubcoreMesh(
            core_axis_name="c", subcore_axis_name="s",
            num_cores=_NC, num_subcores=_NS),
        scratch_types=[
            pltpu.VMEM((_RPW,), jnp.int32),
            pltpu.VMEM((_RPW,), jnp.int32),
            pltpu.VMEM((_RPW,), jnp.float32),
            pltpu.SemaphoreType.DMA,
        ],
    )(_sc_phi_body)


_BCLS = 2000


def _tc_body(s_ref, x_ref, lbl_ref, phi_ref, o_ref):
    cls = pl.program_id(0) * _BCLS + lax.broadcasted_iota(
        jnp.int32, (_BCLS, _ROWS), 0)
    mask = cls == lbl_ref[...]
    o_ref[...] = jnp.where(mask, phi_ref[...], x_ref[...]) * s_ref[0]


def kernel(cosine, labels, s):
    lbl = labels.astype(jnp.int32)
    ct = cosine.T
    flat_phys = jnp.transpose(
        ct.reshape(_COLS // 8, 8, _ROWS // 128, 128), (0, 2, 1, 3)
    ).reshape(-1)
    phi = _sc_phi()(flat_phys, lbl)
    s_arr = jnp.asarray(s, jnp.float32).reshape(1)
    out_t = pl.pallas_call(
        _tc_body,
        grid=(_COLS // _BCLS,),
        in_specs=[
            pl.BlockSpec(memory_space=pltpu.SMEM),
            pl.BlockSpec((_BCLS, _ROWS), lambda i: (i, 0)),
            pl.BlockSpec((1, _ROWS), lambda i: (0, 0)),
            pl.BlockSpec((1, _ROWS), lambda i: (0, 0)),
        ],
        out_specs=pl.BlockSpec((_BCLS, _ROWS), lambda i: (i, 0)),
        out_shape=jax.ShapeDtypeStruct((_COLS, _ROWS), jnp.float32),
        compiler_params=pltpu.CompilerParams(
            dimension_semantics=("arbitrary",)),
    )(s_arr, ct, lbl.reshape(1, _ROWS), phi.reshape(1, _ROWS))
    return out_t.T
